# Initial kernel scaffold; baseline (speedup 1.0000x reference)
import jax
import jax.numpy as jnp
from jax.experimental import pallas as pl


def kernel(x, edge_index, W1, att_src1, att_dst1, b1, W2, att_src2, att_dst2, b2):
    raise NotImplementedError("write your pallas kernel here")


# speedup vs baseline: 2.8833x; 2.8833x over previous
import functools

import jax
import jax.numpy as jnp
from jax.experimental import pallas as pl

N = 10000
E = 320000
IN_DIM = 128
HEADS = 8
HID = 64
OUT = 64
NEG_SLOPE = 0.2

ROWS = 1000


def _proj_attn_kernel(x_ref, w_ref, asrc_ref, adst_ref, h_ref, as_o, ad_o,
                      heads, dim):
    h = jnp.dot(x_ref[...], w_ref[...], preferred_element_type=jnp.float32)
    h_ref[...] = h
    asrc = asrc_ref[...]
    adst = adst_ref[...]
    s_cols = []
    d_cols = []
    for hd in range(heads):
        blk = h[:, hd * dim:(hd + 1) * dim]
        s_cols.append(jnp.sum(blk * asrc[:, hd * dim:(hd + 1) * dim], axis=1,
                              keepdims=True))
        d_cols.append(jnp.sum(blk * adst[:, hd * dim:(hd + 1) * dim], axis=1,
                              keepdims=True))
    as_o[...] = jnp.concatenate(s_cols, axis=1)
    ad_o[...] = jnp.concatenate(d_cols, axis=1)


def _proj_attn(x, w, att_src, att_dst, heads, dim):
    n, in_dim = x.shape
    grid = n // ROWS
    out_shapes = (
        jax.ShapeDtypeStruct((n, heads * dim), jnp.float32),
        jax.ShapeDtypeStruct((n, heads), jnp.float32),
        jax.ShapeDtypeStruct((n, heads), jnp.float32),
    )
    return pl.pallas_call(
        functools.partial(_proj_attn_kernel, heads=heads, dim=dim),
        grid=(grid,),
        in_specs=[
            pl.BlockSpec((ROWS, in_dim), lambda i: (i, 0)),
            pl.BlockSpec((in_dim, heads * dim), lambda i: (0, 0)),
            pl.BlockSpec((1, heads * dim), lambda i: (0, 0)),
            pl.BlockSpec((1, heads * dim), lambda i: (0, 0)),
        ],
        out_specs=(
            pl.BlockSpec((ROWS, heads * dim), lambda i: (i, 0)),
            pl.BlockSpec((ROWS, heads), lambda i: (i, 0)),
            pl.BlockSpec((ROWS, heads), lambda i: (i, 0)),
        ),
        out_shape=out_shapes,
    )(x, w, att_src.reshape(1, heads * dim), att_dst.reshape(1, heads * dim))


def _elu_epilogue_kernel(agg_ref, denom_ref, b_ref, o_ref, heads, dim):
    agg = agg_ref[...]
    denom = denom_ref[...]
    cols = []
    for hd in range(heads):
        cols.append(agg[:, hd * dim:(hd + 1) * dim] / denom[:, hd:hd + 1])
    y = jnp.concatenate(cols, axis=1) + b_ref[...]
    o_ref[...] = jnp.where(y > 0, y, jnp.exp(jnp.minimum(y, 0.0)) - 1.0)


def _elu_epilogue(agg, denom, b, heads, dim):
    n = agg.shape[0]
    return pl.pallas_call(
        functools.partial(_elu_epilogue_kernel, heads=heads, dim=dim),
        grid=(n // ROWS,),
        in_specs=[
            pl.BlockSpec((ROWS, heads * dim), lambda i: (i, 0)),
            pl.BlockSpec((ROWS, heads), lambda i: (i, 0)),
            pl.BlockSpec((1, heads * dim), lambda i: (0, 0)),
        ],
        out_specs=pl.BlockSpec((ROWS, heads * dim), lambda i: (i, 0)),
        out_shape=jax.ShapeDtypeStruct((n, heads * dim), jnp.float32),
    )(agg, denom, b.reshape(1, heads * dim))


def _final_epilogue_kernel(agg_ref, denom_ref, b_ref, o_ref):
    y = agg_ref[...] / denom_ref[...] + b_ref[...]
    m = jnp.max(y, axis=1, keepdims=True)
    z = y - m
    lse = jnp.log(jnp.sum(jnp.exp(z), axis=1, keepdims=True))
    o_ref[...] = z - lse


def _final_epilogue(agg, denom, b):
    n = agg.shape[0]
    d = agg.shape[1]
    return pl.pallas_call(
        _final_epilogue_kernel,
        grid=(n // ROWS,),
        in_specs=[
            pl.BlockSpec((ROWS, d), lambda i: (i, 0)),
            pl.BlockSpec((ROWS, 1), lambda i: (i, 0)),
            pl.BlockSpec((1, d), lambda i: (0, 0)),
        ],
        out_specs=pl.BlockSpec((ROWS, d), lambda i: (i, 0)),
        out_shape=jax.ShapeDtypeStruct((n, d), jnp.float32),
    )(agg, denom.reshape(n, 1), b.reshape(1, d))


def _edge_phase(h, asrc, adst, src, dst, n):
    e = asrc[src] + adst[dst]
    e = jnp.where(e > 0, e, NEG_SLOPE * e)
    e_max = jax.ops.segment_max(e, dst, num_segments=n)
    p = jnp.exp(e - e_max[dst])
    denom = jax.ops.segment_sum(p, dst, num_segments=n)
    heads = asrc.shape[1]
    dim = h.shape[1] // heads
    msg = (h[src].reshape(-1, heads, dim) * p[:, :, None]).reshape(
        -1, heads * dim)
    agg = jax.ops.segment_sum(msg, dst, num_segments=n)
    return agg, denom


def kernel(x, edge_index, W1, att_src1, att_dst1, b1, W2, att_src2, att_dst2,
           b2):
    n = x.shape[0]
    loop = jnp.arange(n, dtype=edge_index.dtype)
    src = jnp.concatenate([edge_index[0], loop])
    dst = jnp.concatenate([edge_index[1], loop])

    h1, asrc1, adst1 = _proj_attn(x, W1, att_src1, att_dst1, HEADS, HID)
    agg1, denom1 = _edge_phase(h1, asrc1, adst1, src, dst, n)
    h = _elu_epilogue(agg1, denom1, b1, HEADS, HID)

    h2, asrc2, adst2 = _proj_attn(h, W2, att_src2, att_dst2, 1, OUT)
    agg2, denom2 = _edge_phase(h2, asrc2, adst2, src, dst, n)
    return _final_epilogue(agg2, denom2, b2)
